# baseline (speedup 1.0000x reference)
import functools
import jax
import jax.numpy as jnp
from jax import lax
from jax.experimental import pallas as pl
from jax.experimental.pallas import tpu as pltpu
from jax.experimental.pallas import tpu_sc as plsc

B = 16384
F = 26
V = 100000
D = 13
E = 16

NC = 2
NS = 16
NW = NC * NS
ROWS_W = B // NW
CH = 128
NCHUNK = ROWS_W // CH
VPC = CH // 16


def _fm_kernel(idx_hbm, t2_hbm, t1_hbm, dn_hbm, wd_hbm, pr_hbm,
               fin_hbm, like_hbm,
               ibuf, gall, wbuf, dbuf, obf, obl, wdv, prm, sem):
    wid = lax.axis_index("s") * NC + lax.axis_index("c")
    row0 = wid * ROWS_W

    pltpu.sync_copy(wd_hbm, wdv)
    pltpu.sync_copy(pr_hbm, prm)
    wd_reg = wdv[...]
    pv = prm[...]
    b_dense = pv[0]
    w_fin = pv[1]
    b_fin = pv[2]
    w_like = pv[3]
    b_like = pv[4]
    zeros16 = jnp.zeros((16,), jnp.float32)

    def chunk_body(c, _):
        rb = row0 + c * CH
        pltpu.sync_copy(idx_hbm.at[:, pl.ds(rb, CH)], ibuf)
        pltpu.sync_copy(dn_hbm.at[:, pl.ds(rb, CH)], dbuf)
        descs = []
        for f in range(F):
            for l in range(16):
                descs.append(pltpu.async_copy(
                    t2_hbm.at[f, l].at[ibuf.at[f]], gall.at[f, l], sem))
            descs.append(pltpu.async_copy(
                t1_hbm.at[f].at[ibuf.at[f]], wbuf.at[f], sem))
        for dsc in descs:
            dsc.wait()

        def comp(v, _):
            vb = v * 16
            so = None
            for l in range(16):
                g = gall[0, l, pl.ds(vb, 16)]
                s = g
                q = g * g
                for f in range(1, F):
                    g = gall[f, l, pl.ds(vb, 16)]
                    s = s + g
                    q = q + g * g
                d = s * s - q
                so = d if so is None else so + d
            dd = dbuf[0, pl.ds(vb, 16)] * wd_reg[0]
            for k in range(1, D):
                dd = dd + dbuf[k, pl.ds(vb, 16)] * wd_reg[k]
            wv = wbuf[0, pl.ds(vb, 16)]
            for f in range(1, F):
                wv = wv + wbuf[f, pl.ds(vb, 16)]
            z = so * 0.5 + wv + dd + b_dense
            fz = 1.0 / (1.0 + jnp.exp(-(z * w_fin + b_fin)))
            lz = 1.0 / (1.0 + jnp.exp(-(z * w_like + b_like)))
            obf[pl.ds(c * CH + vb, 16)] = fz
            obl[pl.ds(c * CH + vb, 16)] = lz
            return 0

        lax.fori_loop(0, VPC, comp, 0)
        return 0

    lax.fori_loop(0, NCHUNK, chunk_body, 0)
    pltpu.sync_copy(obf, fin_hbm.at[pl.ds(row0, ROWS_W)])
    pltpu.sync_copy(obl, like_hbm.at[pl.ds(row0, ROWS_W)])


@functools.partial(
    pl.kernel,
    out_type=(jax.ShapeDtypeStruct((B,), jnp.float32),
              jax.ShapeDtypeStruct((B,), jnp.float32)),
    mesh=plsc.VectorSubcoreMesh(
        core_axis_name="c", subcore_axis_name="s",
        num_cores=NC, num_subcores=NS),
    compiler_params=pltpu.CompilerParams(
        needs_layout_passes=False, use_tc_tiling_on_sc=False),
    scratch_types=(
        pltpu.VMEM((F, CH), jnp.int32),
        pltpu.VMEM((F, 16, CH), jnp.float32),
        pltpu.VMEM((F, CH), jnp.float32),
        pltpu.VMEM((D, CH), jnp.float32),
        pltpu.VMEM((ROWS_W,), jnp.float32),
        pltpu.VMEM((ROWS_W,), jnp.float32),
        pltpu.VMEM((16,), jnp.float32),
        pltpu.VMEM((16,), jnp.float32),
        pltpu.SemaphoreType.DMA,
    ),
)
def _fm_call(*refs):
    _fm_kernel(*refs)


def kernel(sparse_inputs, dense_inputs, W_dense, b_dense, first_tables,
           second_tables, W_finish, b_finish, W_like, b_like):
    idxT = sparse_inputs.T.astype(jnp.int32)
    t2 = second_tables.transpose(0, 2, 1)
    t1 = first_tables.reshape(F, V)
    dn = dense_inputs.T
    wd = jnp.pad(W_dense[:, 0], (0, 16 - D))
    prm = jnp.concatenate([
        b_dense, W_finish[0], b_finish, W_like[0], b_like,
        jnp.zeros((11,), jnp.float32)])
    fin, like = _fm_call(idxT, t2, t1, dn, wd, prm)
    return (fin.reshape(B, 1), like.reshape(B, 1))


# speedup vs baseline: 1.0003x; 1.0003x over previous
import functools
import jax
import jax.numpy as jnp
from jax import lax
from jax.experimental import pallas as pl
from jax.experimental.pallas import tpu as pltpu
from jax.experimental.pallas import tpu_sc as plsc

B = 16384
F = 26
V = 100000
D = 13
E = 16

NC = 2
NS = 16
NW = NC * NS
ROWS_W = B // NW
CH = 128
NCHUNK = ROWS_W // CH
VPC = CH // 16


def _fm_kernel(idx_hbm, t2_hbm, t1_hbm, dn_hbm, wd_hbm, pr_hbm,
               fin_hbm, like_hbm,
               ibuf, gall, wbuf, dbuf, obf, obl, wdv, prm, sem):
    wid = lax.axis_index("s") * NC + lax.axis_index("c")
    row0 = wid * ROWS_W

    pltpu.sync_copy(wd_hbm, wdv)
    pltpu.sync_copy(pr_hbm, prm)
    wd_reg = wdv[...]
    pv = prm[...]
    b_dense = pv[0]
    w_fin = pv[1]
    b_fin = pv[2]
    w_like = pv[3]
    b_like = pv[4]

    def chunk_body(c, _):
        rb = row0 + c * CH
        pltpu.sync_copy(idx_hbm.at[:, pl.ds(rb, CH)], ibuf)
        pltpu.sync_copy(dn_hbm.at[:, pl.ds(rb, CH)], dbuf)
        descs = []
        for f in range(F):
            for l in range(16):
                descs.append(pltpu.async_copy(
                    t2_hbm.at[f, l].at[ibuf.at[f]], gall.at[f, l], sem))
            descs.append(pltpu.async_copy(
                t1_hbm.at[f].at[ibuf.at[f]], wbuf.at[f], sem))
        for dsc in descs:
            dsc.wait()

        def comp(v, _):
            vb = v * 16
            so = None
            for l in range(16):
                g = gall[0, l, pl.ds(vb, 16)]
                s = g
                q = g * g
                for f in range(1, F):
                    g = gall[f, l, pl.ds(vb, 16)]
                    s = s + g
                    q = q + g * g
                d = s * s - q
                so = d if so is None else so + d
            dd = dbuf[0, pl.ds(vb, 16)] * wd_reg[0]
            for k in range(1, D):
                dd = dd + dbuf[k, pl.ds(vb, 16)] * wd_reg[k]
            wv = wbuf[0, pl.ds(vb, 16)]
            for f in range(1, F):
                wv = wv + wbuf[f, pl.ds(vb, 16)]
            z = so * 0.5 + wv + dd + b_dense
            fz = 1.0 / (1.0 + jnp.exp(-(z * w_fin + b_fin)))
            lz = 1.0 / (1.0 + jnp.exp(-(z * w_like + b_like)))
            obf[pl.ds(c * CH + vb, 16)] = fz
            obl[pl.ds(c * CH + vb, 16)] = lz
            return 0

        lax.fori_loop(0, VPC, comp, 0)
        return 0

    lax.fori_loop(0, NCHUNK, chunk_body, 0)
    pltpu.sync_copy(obf, fin_hbm.at[pl.ds(row0, ROWS_W)])
    pltpu.sync_copy(obl, like_hbm.at[pl.ds(row0, ROWS_W)])


@functools.partial(
    pl.kernel,
    out_type=(jax.ShapeDtypeStruct((B,), jnp.float32),
              jax.ShapeDtypeStruct((B,), jnp.float32)),
    mesh=plsc.VectorSubcoreMesh(
        core_axis_name="c", subcore_axis_name="s",
        num_cores=NC, num_subcores=NS),
    compiler_params=pltpu.CompilerParams(
        needs_layout_passes=False, use_tc_tiling_on_sc=False),
    scratch_types=(
        pltpu.VMEM((F, CH), jnp.int32),
        pltpu.VMEM((F, 16, CH), jnp.float32),
        pltpu.VMEM((F, CH), jnp.float32),
        pltpu.VMEM((D, CH), jnp.float32),
        pltpu.VMEM((ROWS_W,), jnp.float32),
        pltpu.VMEM((ROWS_W,), jnp.float32),
        pltpu.VMEM((16,), jnp.float32),
        pltpu.VMEM((16,), jnp.float32),
        pltpu.SemaphoreType.DMA,
    ),
)
def _fm_call(*refs):
    _fm_kernel(*refs)


def kernel(sparse_inputs, dense_inputs, W_dense, b_dense, first_tables,
           second_tables, W_finish, b_finish, W_like, b_like):
    idxT = sparse_inputs.T.astype(jnp.int32)
    t2 = second_tables.transpose(0, 2, 1)
    t1 = first_tables.reshape(F, V)
    dn = dense_inputs.T
    wd = jnp.pad(W_dense[:, 0], (0, 16 - D))
    prm = jnp.concatenate([
        b_dense, W_finish[0], b_finish, W_like[0], b_like,
        jnp.zeros((11,), jnp.float32)])
    fin, like = _fm_call(idxT, t2, t1, dn, wd, prm)
    return (fin.reshape(B, 1), like.reshape(B, 1))
